# baseline (speedup 1.0000x reference)
import functools

import jax
import jax.numpy as jnp
from jax import lax
from jax.experimental import pallas as pl
from jax.experimental.pallas import tpu as pltpu
from jax.experimental.pallas import tpu_sc as plsc

_CHUNK = 64
_NBUF = 2


def kernel(x, emb_weight):
    sl = x.shape[1]
    dim = emb_weight.shape[1]
    info = plsc.get_sparse_core_info()
    nc, ns = info.num_cores, info.num_subcores
    nw = nc * ns
    nchunks = sl // (_CHUNK * nw)

    mesh = plsc.VectorSubcoreMesh(core_axis_name="c", subcore_axis_name="s")

    @functools.partial(
        pl.kernel,
        mesh=mesh,
        out_type=jax.ShapeDtypeStruct((sl, dim), emb_weight.dtype),
        scratch_types=(
            [pltpu.VMEM((_CHUNK, dim), jnp.float32) for _ in range(_NBUF)]
            + [pltpu.SemaphoreType.DMA for _ in range(2 * _NBUF)]
        ),
    )
    def copy_k(emb_hbm, out_hbm, *scratch):
        bufs = scratch[:_NBUF]
        isems = scratch[_NBUF : 2 * _NBUF]
        osems = scratch[2 * _NBUF :]
        wid = lax.axis_index("s") * nc + lax.axis_index("c")

        def row0(i):
            return (i * nw + wid) * _CHUNK

        def load(i):
            b = i % _NBUF
            return pltpu.async_copy(
                emb_hbm.at[pl.ds(row0(i), _CHUNK)], bufs[b], isems[b]
            )

        def store(i):
            b = i % _NBUF
            return pltpu.async_copy(
                bufs[b], out_hbm.at[pl.ds(row0(i), _CHUNK)], osems[b]
            )

        loads = {}
        stores = {}
        for i in range(min(_NBUF, nchunks)):
            loads[i] = load(i)
        for i in range(nchunks):
            if i >= _NBUF:
                stores[i - _NBUF].wait()
                loads[i] = load(i)
            loads[i].wait()
            stores[i] = store(i)
        for i in range(max(0, nchunks - _NBUF), nchunks):
            stores[i].wait()

    return copy_k(emb_weight)


# speedup vs baseline: 1.0300x; 1.0300x over previous
import functools

import jax
import jax.numpy as jnp
from jax import lax
from jax.experimental import pallas as pl
from jax.experimental.pallas import tpu as pltpu
from jax.experimental.pallas import tpu_sc as plsc

_CHUNK = 64
_NBUF = 2
_TC_BLOCK = 2048


def _sc_copy_bottom(emb_weight, sl, dim, lo):
    info = plsc.get_sparse_core_info()
    nc, ns = info.num_cores, info.num_subcores
    nw = nc * ns
    rows_per_w = (sl - lo) // nw
    nchunks = rows_per_w // _CHUNK

    mesh = plsc.VectorSubcoreMesh(core_axis_name="c", subcore_axis_name="s")

    @functools.partial(
        pl.kernel,
        mesh=mesh,
        out_type=jax.ShapeDtypeStruct((sl, dim), emb_weight.dtype),
        scratch_types=(
            [pltpu.VMEM((_CHUNK, dim), jnp.float32) for _ in range(_NBUF)]
            + [pltpu.SemaphoreType.DMA for _ in range(2 * _NBUF)]
        ),
    )
    def copy_k(emb_hbm, out_hbm, *scratch):
        bufs = scratch[:_NBUF]
        isems = scratch[_NBUF : 2 * _NBUF]
        osems = scratch[2 * _NBUF :]
        wid = lax.axis_index("s") * nc + lax.axis_index("c")
        base = lo + wid * rows_per_w

        def load(i):
            b = i % _NBUF
            return pltpu.async_copy(
                emb_hbm.at[pl.ds(base + i * _CHUNK, _CHUNK)], bufs[b], isems[b]
            )

        def store(i):
            b = i % _NBUF
            return pltpu.async_copy(
                bufs[b], out_hbm.at[pl.ds(base + i * _CHUNK, _CHUNK)], osems[b]
            )

        loads = {}
        stores = {}
        for i in range(min(_NBUF, nchunks)):
            loads[i] = load(i)
        for i in range(nchunks):
            if i >= _NBUF:
                stores[i - _NBUF].wait()
                loads[i] = load(i)
            loads[i].wait()
            stores[i] = store(i)
        for i in range(max(0, nchunks - _NBUF), nchunks):
            stores[i].wait()

    return copy_k(emb_weight)


def _tc_body(in_ref, partial_any, out_ref):
    del partial_any
    out_ref[...] = in_ref[...]


def _tc_copy_top(emb_weight, partial_out, sl, dim, hi):
    return pl.pallas_call(
        _tc_body,
        out_shape=jax.ShapeDtypeStruct((sl, dim), emb_weight.dtype),
        grid=(hi // _TC_BLOCK,),
        in_specs=[
            pl.BlockSpec((_TC_BLOCK, dim), lambda i: (i, 0)),
            pl.BlockSpec(memory_space=pl.ANY),
        ],
        out_specs=pl.BlockSpec((_TC_BLOCK, dim), lambda i: (i, 0)),
        input_output_aliases={1: 0},
    )(emb_weight, partial_out)


def kernel(x, emb_weight):
    sl = x.shape[1]
    dim = emb_weight.shape[1]
    split = sl // 2
    partial = _sc_copy_bottom(emb_weight, sl, dim, split)
    return _tc_copy_top(emb_weight, partial, sl, dim, split)


# speedup vs baseline: 1.0384x; 1.0082x over previous
import functools

import jax
import jax.numpy as jnp
from jax import lax
from jax.experimental import pallas as pl
from jax.experimental.pallas import tpu as pltpu
from jax.experimental.pallas import tpu_sc as plsc

_CHUNK = 48
_NBUF = 2
_TC_BLOCK = 2560


def _sc_copy_bottom(emb_weight, sl, dim, lo):
    info = plsc.get_sparse_core_info()
    nc, ns = info.num_cores, info.num_subcores
    nw = nc * ns
    rows_per_w = (sl - lo) // nw
    nchunks = rows_per_w // _CHUNK

    mesh = plsc.VectorSubcoreMesh(core_axis_name="c", subcore_axis_name="s")

    @functools.partial(
        pl.kernel,
        mesh=mesh,
        out_type=jax.ShapeDtypeStruct((sl, dim), emb_weight.dtype),
        scratch_types=(
            [pltpu.VMEM((_CHUNK, dim), jnp.float32) for _ in range(_NBUF)]
            + [pltpu.SemaphoreType.DMA for _ in range(2 * _NBUF)]
        ),
    )
    def copy_k(emb_hbm, out_hbm, *scratch):
        bufs = scratch[:_NBUF]
        isems = scratch[_NBUF : 2 * _NBUF]
        osems = scratch[2 * _NBUF :]
        wid = lax.axis_index("s") * nc + lax.axis_index("c")
        base = lo + wid * rows_per_w

        def load(i):
            b = i % _NBUF
            return pltpu.async_copy(
                emb_hbm.at[pl.ds(base + i * _CHUNK, _CHUNK)], bufs[b], isems[b]
            )

        def store(i):
            b = i % _NBUF
            return pltpu.async_copy(
                bufs[b], out_hbm.at[pl.ds(base + i * _CHUNK, _CHUNK)], osems[b]
            )

        loads = {}
        stores = {}
        for i in range(min(_NBUF, nchunks)):
            loads[i] = load(i)
        for i in range(nchunks):
            if i >= _NBUF:
                stores[i - _NBUF].wait()
                loads[i] = load(i)
            loads[i].wait()
            stores[i] = store(i)
        for i in range(max(0, nchunks - _NBUF), nchunks):
            stores[i].wait()

    return copy_k(emb_weight)


def _tc_body(in_ref, partial_any, out_ref):
    del partial_any
    out_ref[...] = in_ref[...]


def _tc_copy_top(emb_weight, partial_out, sl, dim, hi):
    return pl.pallas_call(
        _tc_body,
        out_shape=jax.ShapeDtypeStruct((sl, dim), emb_weight.dtype),
        grid=(hi // _TC_BLOCK,),
        in_specs=[
            pl.BlockSpec((_TC_BLOCK, dim), lambda i: (i, 0)),
            pl.BlockSpec(memory_space=pl.ANY),
        ],
        out_specs=pl.BlockSpec((_TC_BLOCK, dim), lambda i: (i, 0)),
        input_output_aliases={1: 0},
    )(emb_weight, partial_out)


def kernel(x, emb_weight):
    sl = x.shape[1]
    dim = emb_weight.shape[1]
    split = (sl * 5) // 8
    partial = _sc_copy_bottom(emb_weight, sl, dim, split)
    return _tc_copy_top(emb_weight, partial, sl, dim, split)
